# baseline (speedup 1.0000x reference)
import jax
import jax.numpy as jnp
from jax.experimental import pallas as pl

_MAX_LENGTH = 128
_BLOCK_M = 3584


def _select_body(a_ref, o_ref):
    a = a_ref[...]
    t = a_ref.shape[1]
    l = o_ref.shape[1]
    idx = (jnp.arange(l, dtype=jnp.int32) * (t - 1)) // (l - 1)
    lane = idx % l
    hi = idx // l
    lane2 = jnp.broadcast_to(lane[None, :], (a.shape[0], l))
    g0 = jnp.take_along_axis(a[:, :l], lane2, axis=1)
    g1 = jnp.take_along_axis(a[:, l:], lane2, axis=1)
    o_ref[...] = jnp.where((hi == 0)[None, :], g0, g1)


def kernel(x):
    T, C, H, W = x.shape
    L = _MAX_LENGTH
    M = C * H * W
    xt = jnp.transpose(x, (1, 2, 3, 0)).reshape(M, T)

    out2 = pl.pallas_call(
        _select_body,
        grid=(M // _BLOCK_M,),
        in_specs=[pl.BlockSpec((_BLOCK_M, T), lambda i: (i, 0))],
        out_specs=pl.BlockSpec((_BLOCK_M, L), lambda i: (i, 0)),
        out_shape=jax.ShapeDtypeStruct((M, L), x.dtype),
    )(xt)
    return jnp.transpose(out2.reshape(C, H, W, L), (3, 0, 1, 2))


# speedup vs baseline: 1.5051x; 1.5051x over previous
import jax
import jax.numpy as jnp
from jax.experimental import pallas as pl

_MAX_LENGTH = 128
_BLOCK_M = 7168


def _select_body(a_ref, o_ref):
    a = a_ref[...]
    t = a_ref.shape[1]
    l = o_ref.shape[1]
    m = a.shape[0]
    j = jax.lax.broadcasted_iota(jnp.int32, (m, l), 1)
    idx = (j * (t - 1)) // (l - 1)
    lane = idx % l
    hi = idx // l
    g0 = jnp.take_along_axis(a[:, :l], lane, axis=1)
    g1 = jnp.take_along_axis(a[:, l:], lane, axis=1)
    o_ref[...] = jnp.where(hi == 0, g0, g1)


def kernel(x):
    T, C, H, W = x.shape
    L = _MAX_LENGTH
    M = C * H * W
    xt = jnp.transpose(x, (1, 2, 3, 0)).reshape(M, T)

    out2 = pl.pallas_call(
        _select_body,
        grid=(M // _BLOCK_M,),
        in_specs=[pl.BlockSpec((_BLOCK_M, T), lambda i: (i, 0))],
        out_specs=pl.BlockSpec((_BLOCK_M, L), lambda i: (i, 0)),
        out_shape=jax.ShapeDtypeStruct((M, L), x.dtype),
    )(xt)
    return jnp.transpose(out2.reshape(C, H, W, L), (3, 0, 1, 2))


# speedup vs baseline: 1.5334x; 1.0188x over previous
import jax
import jax.numpy as jnp
from jax.experimental import pallas as pl

_MAX_LENGTH = 128
_BLOCK_M = 9408


def _select_body(a_ref, o_ref):
    a = a_ref[...]
    t = a_ref.shape[1]
    l = o_ref.shape[1]
    m = a.shape[0]
    j = jax.lax.broadcasted_iota(jnp.int32, (m, l), 1)
    idx = (j * (t - 1)) // (l - 1)
    lane = idx % l
    hi = idx // l
    g0 = jnp.take_along_axis(a[:, :l], lane, axis=1)
    g1 = jnp.take_along_axis(a[:, l:], lane, axis=1)
    o_ref[...] = jnp.where(hi == 0, g0, g1)


def kernel(x):
    T, C, H, W = x.shape
    L = _MAX_LENGTH
    M = C * H * W
    xt = jnp.transpose(x, (1, 2, 3, 0)).reshape(M, T)

    out2 = pl.pallas_call(
        _select_body,
        grid=(M // _BLOCK_M,),
        in_specs=[pl.BlockSpec((_BLOCK_M, T), lambda i: (i, 0))],
        out_specs=pl.BlockSpec((_BLOCK_M, L), lambda i: (i, 0)),
        out_shape=jax.ShapeDtypeStruct((M, L), x.dtype),
    )(xt)
    return jnp.transpose(out2.reshape(C, H, W, L), (3, 0, 1, 2))


# speedup vs baseline: 1.5574x; 1.0156x over previous
import jax
import jax.numpy as jnp
from jax.experimental import pallas as pl

_MAX_LENGTH = 128
_BLOCK_M = 10752


def _select_body(a_ref, o_ref):
    a = a_ref[...]
    t = a_ref.shape[1]
    l = o_ref.shape[1]
    m = a.shape[0]
    j = jax.lax.broadcasted_iota(jnp.int32, (m, l), 1)
    idx = (j * (t - 1)) // (l - 1)
    lane = idx % l
    hi = idx // l
    g0 = jnp.take_along_axis(a[:, :l], lane, axis=1)
    g1 = jnp.take_along_axis(a[:, l:], lane, axis=1)
    o_ref[...] = jnp.where(hi == 0, g0, g1)


def kernel(x):
    T, C, H, W = x.shape
    L = _MAX_LENGTH
    M = C * H * W
    xt = jnp.transpose(x, (1, 2, 3, 0)).reshape(M, T)

    out2 = pl.pallas_call(
        _select_body,
        grid=(M // _BLOCK_M,),
        in_specs=[pl.BlockSpec((_BLOCK_M, T), lambda i: (i, 0))],
        out_specs=pl.BlockSpec((_BLOCK_M, L), lambda i: (i, 0)),
        out_shape=jax.ShapeDtypeStruct((M, L), x.dtype),
    )(xt)
    return jnp.transpose(out2.reshape(C, H, W, L), (3, 0, 1, 2))


# speedup vs baseline: 1.5736x; 1.0105x over previous
import jax
import jax.numpy as jnp
from jax.experimental import pallas as pl

_MAX_LENGTH = 128
_BLOCK_M = 12544


def _select_body(a_ref, o_ref):
    a = a_ref[...]
    t = a_ref.shape[1]
    l = o_ref.shape[1]
    m = a.shape[0]
    j = jax.lax.broadcasted_iota(jnp.int32, (m, l), 1)
    idx = (j * (t - 1)) // (l - 1)
    lane = idx % l
    hi = idx // l
    g0 = jnp.take_along_axis(a[:, :l], lane, axis=1)
    g1 = jnp.take_along_axis(a[:, l:], lane, axis=1)
    o_ref[...] = jnp.where(hi == 0, g0, g1)


def kernel(x):
    T, C, H, W = x.shape
    L = _MAX_LENGTH
    M = C * H * W
    xt = jnp.transpose(x, (1, 2, 3, 0)).reshape(M, T)

    out2 = pl.pallas_call(
        _select_body,
        grid=(M // _BLOCK_M,),
        in_specs=[pl.BlockSpec((_BLOCK_M, T), lambda i: (i, 0))],
        out_specs=pl.BlockSpec((_BLOCK_M, L), lambda i: (i, 0)),
        out_shape=jax.ShapeDtypeStruct((M, L), x.dtype),
    )(xt)
    return jnp.transpose(out2.reshape(C, H, W, L), (3, 0, 1, 2))
